# baseline (speedup 1.0000x reference)
import functools

import jax
import jax.numpy as jnp
from jax import lax
from jax.experimental import pallas as pl
from jax.experimental.pallas import tpu as pltpu
from jax.experimental.pallas import tpu_sc as plsc

_LANES = 16


def _make_sc_kernel(batch, num_fields, nc, ns):
    nw = nc * ns
    n_per = batch // nw
    n_flat = n_per * num_fields

    def body(x_hbm, w_hbm, b_hbm, out_hbm, idx_v, vals_v, out_v, bias_v, sem):
        cid = lax.axis_index("c")
        sid = lax.axis_index("s")
        wid = sid * nc + cid
        base = wid * n_flat

        pltpu.sync_copy(x_hbm.at[pl.ds(base, n_flat)], idx_v)
        pltpu.sync_copy(b_hbm, bias_v.at[pl.ds(0, 1)])

        pltpu.async_copy(w_hbm.at[idx_v], vals_v, sem).wait()

        bias_vec = jnp.broadcast_to(bias_v[pl.ds(0, _LANES)][0], (_LANES,))
        lane_f = lax.iota(jnp.int32, _LANES) * num_fields

        def chunk(c, carry):
            g0 = lane_f + c * (_LANES * num_fields)
            acc = bias_vec
            for j in range(num_fields):
                acc = acc + plsc.load_gather(vals_v, [g0 + j])
            out_v[pl.ds(c * _LANES, _LANES)] = acc
            return carry

        lax.fori_loop(0, n_per // _LANES, chunk, 0)
        pltpu.sync_copy(out_v, out_hbm.at[pl.ds(wid * n_per, n_per)])

    mesh = plsc.VectorSubcoreMesh(
        core_axis_name="c", subcore_axis_name="s", num_cores=nc
    )
    return pl.kernel(
        body,
        out_type=jax.ShapeDtypeStruct((batch,), jnp.float32),
        mesh=mesh,
        compiler_params=pltpu.CompilerParams(
            needs_layout_passes=False, skip_device_barrier=True
        ),
        scratch_types=[
            pltpu.VMEM((n_flat,), jnp.int32),
            pltpu.VMEM((n_flat,), jnp.float32),
            pltpu.VMEM((n_per,), jnp.float32),
            pltpu.VMEM((128,), jnp.float32),
            pltpu.SemaphoreType.DMA,
        ],
    )


@jax.jit
def kernel(x, fc_weight, bias):
    batch, num_fields = x.shape
    info = plsc.get_sparse_core_info()
    nc, ns = 1, info.num_subcores

    x_flat = x.astype(jnp.int32).reshape(-1)
    w_flat = fc_weight.reshape(-1)
    b_flat = bias.reshape(-1).astype(jnp.float32)

    sc = _make_sc_kernel(batch, num_fields, nc, ns)
    out = sc(x_flat, w_flat, b_flat)
    return out.reshape(batch, 1)


# speedup vs baseline: 2.1642x; 2.1642x over previous
import functools

import jax
import jax.numpy as jnp
from jax import lax
from jax.experimental import pallas as pl
from jax.experimental.pallas import tpu as pltpu
from jax.experimental.pallas import tpu_sc as plsc

_LANES = 16


def _make_sc_kernel(batch, num_fields, nc, ns):
    nw = nc * ns
    n_per = batch // nw

    def body(xt_hbm, w_hbm, b_hbm, out_hbm, *scr):
        idxs = scr[:num_fields]
        vals = scr[num_fields:2 * num_fields]
        out_v, bias_v, sem = scr[2 * num_fields:]
        cid = lax.axis_index("c")
        sid = lax.axis_index("s")
        wid = sid * nc + cid
        b0 = wid * n_per

        for j in range(num_fields):
            pltpu.sync_copy(xt_hbm.at[j, pl.ds(b0, n_per)], idxs[j])
        pltpu.sync_copy(b_hbm, bias_v.at[pl.ds(0, 1)])

        descs = [
            pltpu.async_copy(w_hbm.at[0].at[idxs[j]], vals[j], sem)
            for j in range(num_fields)
        ]
        for d in descs:
            d.wait()

        bias_vec = jnp.broadcast_to(bias_v[pl.ds(0, _LANES)][0], (_LANES,))

        def chunk(c, carry):
            acc = bias_vec
            for j in range(num_fields):
                acc = acc + vals[j][pl.ds(c * _LANES, _LANES)]
            out_v[pl.ds(c * _LANES, _LANES)] = acc
            return carry

        lax.fori_loop(0, n_per // _LANES, chunk, 0)
        pltpu.sync_copy(out_v, out_hbm.at[pl.ds(b0, n_per)])

    mesh = plsc.VectorSubcoreMesh(
        core_axis_name="c", subcore_axis_name="s", num_cores=nc
    )
    return pl.kernel(
        body,
        out_type=jax.ShapeDtypeStruct((batch,), jnp.float32),
        mesh=mesh,
        compiler_params=pltpu.CompilerParams(
            needs_layout_passes=False, skip_device_barrier=True
        ),
        scratch_types=(
            [pltpu.VMEM((n_per,), jnp.int32) for _ in range(num_fields)]
            + [pltpu.VMEM((n_per,), jnp.float32) for _ in range(num_fields)]
            + [
                pltpu.VMEM((n_per,), jnp.float32),
                pltpu.VMEM((128,), jnp.float32),
                pltpu.SemaphoreType.DMA,
            ]
        ),
    )


@jax.jit
def kernel(x, fc_weight, bias):
    batch, num_fields = x.shape
    info = plsc.get_sparse_core_info()
    nc, ns = info.num_cores, info.num_subcores

    sc = _make_sc_kernel(batch, num_fields, nc, ns)
    out = sc(x.T.astype(jnp.int32), fc_weight.T, bias.astype(jnp.float32))
    return out.reshape(batch, 1)


# speedup vs baseline: 2.8976x; 1.3389x over previous
import functools

import jax
import jax.numpy as jnp
from jax import lax
from jax.experimental import pallas as pl
from jax.experimental.pallas import tpu as pltpu
from jax.experimental.pallas import tpu_sc as plsc

_LANES = 16


def _make_sc_kernel(batch, num_fields, nc, ns):
    nw = nc * ns
    n_per = batch // nw

    n_flat = n_per * num_fields

    def body(xt_hbm, w_hbm, b_hbm, out_hbm, idx_v, vals_v, out_v, bias_v,
             sem, sem2):
        cid = lax.axis_index("c")
        sid = lax.axis_index("s")
        wid = sid * nc + cid
        b0 = wid * n_per

        descs = [
            pltpu.async_copy(
                xt_hbm.at[j, pl.ds(b0, n_per)],
                idx_v.at[pl.ds(j * n_per, n_per)],
                sem2,
            )
            for j in range(num_fields)
        ]
        pltpu.sync_copy(b_hbm, bias_v.at[pl.ds(0, 1)])
        for d in descs:
            d.wait()

        pltpu.async_copy(w_hbm.at[0].at[idx_v], vals_v, sem).wait()

        bias_vec = jnp.broadcast_to(bias_v[pl.ds(0, _LANES)][0], (_LANES,))

        def chunk(c, carry):
            acc = bias_vec
            for j in range(num_fields):
                acc = acc + vals_v[pl.ds(j * n_per + c * _LANES, _LANES)]
            out_v[pl.ds(c * _LANES, _LANES)] = acc
            return carry

        lax.fori_loop(0, n_per // _LANES, chunk, 0)
        pltpu.sync_copy(out_v, out_hbm.at[pl.ds(b0, n_per)])

    mesh = plsc.VectorSubcoreMesh(
        core_axis_name="c", subcore_axis_name="s", num_cores=nc
    )
    return pl.kernel(
        body,
        out_type=jax.ShapeDtypeStruct((batch,), jnp.float32),
        mesh=mesh,
        compiler_params=pltpu.CompilerParams(
            needs_layout_passes=False, skip_device_barrier=True
        ),
        scratch_types=[
            pltpu.VMEM((n_flat,), jnp.int32),
            pltpu.VMEM((n_flat,), jnp.float32),
            pltpu.VMEM((n_per,), jnp.float32),
            pltpu.VMEM((128,), jnp.float32),
            pltpu.SemaphoreType.DMA,
            pltpu.SemaphoreType.DMA,
        ],
    )


@jax.jit
def kernel(x, fc_weight, bias):
    batch, num_fields = x.shape
    info = plsc.get_sparse_core_info()
    nc, ns = info.num_cores, info.num_subcores

    sc = _make_sc_kernel(batch, num_fields, nc, ns)
    out = sc(x.T.astype(jnp.int32), fc_weight.T, bias.astype(jnp.float32))
    return out.reshape(batch, 1)


# speedup vs baseline: 2.9155x; 1.0062x over previous
import functools

import jax
import jax.numpy as jnp
from jax import lax
from jax.experimental import pallas as pl
from jax.experimental.pallas import tpu as pltpu
from jax.experimental.pallas import tpu_sc as plsc

_LANES = 16


def _make_sc_kernel(batch, num_fields, nc, ns):
    nw = nc * ns
    n_per = batch // nw

    n_flat = n_per * num_fields

    def body(xt_hbm, w_hbm, b_hbm, out_hbm, idx_v, vals_v, out_v, bias_v,
             sem, sem2, sem3):
        cid = lax.axis_index("c")
        sid = lax.axis_index("s")
        wid = sid * nc + cid
        b0 = wid * n_per

        descs = [
            pltpu.async_copy(
                xt_hbm.at[j, pl.ds(b0, n_per)],
                idx_v.at[pl.ds(j * n_per, n_per)],
                sem2 if j < num_fields // 2 else sem3,
            )
            for j in range(num_fields)
        ]
        pltpu.sync_copy(b_hbm, bias_v.at[pl.ds(0, 1)])

        half = num_fields // 2
        cut = half * n_per
        for d in descs[:half]:
            d.wait()
        ga = pltpu.async_copy(
            w_hbm.at[0].at[idx_v.at[pl.ds(0, cut)]],
            vals_v.at[pl.ds(0, cut)], sem)
        for d in descs[half:]:
            d.wait()
        gb = pltpu.async_copy(
            w_hbm.at[0].at[idx_v.at[pl.ds(cut, n_flat - cut)]],
            vals_v.at[pl.ds(cut, n_flat - cut)], sem3)

        bias_vec = jnp.broadcast_to(bias_v[pl.ds(0, _LANES)][0], (_LANES,))
        nchunk = n_per // _LANES

        ga.wait()

        def chunk_a(c, carry):
            acc = bias_vec
            for j in range(half):
                acc = acc + vals_v[pl.ds(j * n_per + c * _LANES, _LANES)]
            out_v[pl.ds(c * _LANES, _LANES)] = acc
            return carry

        lax.fori_loop(0, nchunk, chunk_a, 0)
        gb.wait()

        def chunk_b(c, carry):
            acc = out_v[pl.ds(c * _LANES, _LANES)]
            for j in range(half, num_fields):
                acc = acc + vals_v[pl.ds(j * n_per + c * _LANES, _LANES)]
            out_v[pl.ds(c * _LANES, _LANES)] = acc
            return carry

        lax.fori_loop(0, nchunk, chunk_b, 0)
        pltpu.sync_copy(out_v, out_hbm.at[pl.ds(b0, n_per)])

    mesh = plsc.VectorSubcoreMesh(
        core_axis_name="c", subcore_axis_name="s", num_cores=nc
    )
    return pl.kernel(
        body,
        out_type=jax.ShapeDtypeStruct((batch,), jnp.float32),
        mesh=mesh,
        compiler_params=pltpu.CompilerParams(
            needs_layout_passes=False, skip_device_barrier=True
        ),
        scratch_types=[
            pltpu.VMEM((n_flat,), jnp.int32),
            pltpu.VMEM((n_flat,), jnp.float32),
            pltpu.VMEM((n_per,), jnp.float32),
            pltpu.VMEM((128,), jnp.float32),
            pltpu.SemaphoreType.DMA,
            pltpu.SemaphoreType.DMA,
            pltpu.SemaphoreType.DMA,
        ],
    )


@jax.jit
def kernel(x, fc_weight, bias):
    batch, num_fields = x.shape
    info = plsc.get_sparse_core_info()
    nc, ns = info.num_cores, info.num_subcores

    sc = _make_sc_kernel(batch, num_fields, nc, ns)
    out = sc(x.T.astype(jnp.int32), fc_weight.T, bias.astype(jnp.float32))
    return out.reshape(batch, 1)


# speedup vs baseline: 3.4313x; 1.1769x over previous
import functools

import jax
import jax.numpy as jnp
from jax import lax
from jax.experimental import pallas as pl
from jax.experimental.pallas import tpu as pltpu
from jax.experimental.pallas import tpu_sc as plsc

_LANES = 16


def _make_sc_kernel(batch, num_fields, vocab, nc, ns):
    nw = nc * ns
    n_per = batch // nw

    n_flat = n_per * num_fields

    def body(xt_hbm, w_hbm, b_hbm, out_hbm, idx_v, vals_v, out_v, bias_v,
             w_sp, sem, sem2, sem3, semw):
        cid = lax.axis_index("c")
        sid = lax.axis_index("s")
        wid = sid * nc + cid
        b0 = wid * n_per

        descs = [
            pltpu.async_copy(
                xt_hbm.at[j, pl.ds(b0, n_per)],
                idx_v.at[pl.ds(j * n_per, n_per)],
                sem2 if j < num_fields // 2 else sem3,
            )
            for j in range(num_fields)
        ]
        pltpu.sync_copy(b_hbm, bias_v.at[pl.ds(0, 1)])

        @pl.when(sid == 0)
        def _():
            pltpu.async_copy(w_hbm, w_sp, semw).wait()

        for d in descs:
            d.wait()
        plsc.subcore_barrier()

        half = num_fields // 2
        cut = half * n_per
        ga = pltpu.async_copy(
            w_sp.at[0].at[idx_v.at[pl.ds(0, cut)]],
            vals_v.at[pl.ds(0, cut)], sem)
        gb = pltpu.async_copy(
            w_sp.at[0].at[idx_v.at[pl.ds(cut, n_flat - cut)]],
            vals_v.at[pl.ds(cut, n_flat - cut)], sem3)

        bias_vec = jnp.broadcast_to(bias_v[pl.ds(0, _LANES)][0], (_LANES,))
        nchunk = n_per // _LANES

        ga.wait()

        def chunk_a(c, carry):
            acc = bias_vec
            for j in range(half):
                acc = acc + vals_v[pl.ds(j * n_per + c * _LANES, _LANES)]
            out_v[pl.ds(c * _LANES, _LANES)] = acc
            return carry

        lax.fori_loop(0, nchunk, chunk_a, 0)
        gb.wait()

        def chunk_b(c, carry):
            acc = out_v[pl.ds(c * _LANES, _LANES)]
            for j in range(half, num_fields):
                acc = acc + vals_v[pl.ds(j * n_per + c * _LANES, _LANES)]
            out_v[pl.ds(c * _LANES, _LANES)] = acc
            return carry

        lax.fori_loop(0, nchunk, chunk_b, 0)
        pltpu.sync_copy(out_v, out_hbm.at[pl.ds(b0, n_per)])

    mesh = plsc.VectorSubcoreMesh(
        core_axis_name="c", subcore_axis_name="s", num_cores=nc
    )
    return pl.kernel(
        body,
        out_type=jax.ShapeDtypeStruct((batch,), jnp.float32),
        mesh=mesh,
        compiler_params=pltpu.CompilerParams(
            needs_layout_passes=False, skip_device_barrier=True
        ),
        scratch_types=[
            pltpu.VMEM((n_flat,), jnp.int32),
            pltpu.VMEM((n_flat,), jnp.float32),
            pltpu.VMEM((n_per,), jnp.float32),
            pltpu.VMEM((128,), jnp.float32),
            pltpu.VMEM_SHARED((1, vocab), jnp.float32),
            pltpu.SemaphoreType.DMA,
            pltpu.SemaphoreType.DMA,
            pltpu.SemaphoreType.DMA,
            pltpu.SemaphoreType.DMA,
        ],
    )


@jax.jit
def kernel(x, fc_weight, bias):
    batch, num_fields = x.shape
    info = plsc.get_sparse_core_info()
    nc, ns = info.num_cores, info.num_subcores

    sc = _make_sc_kernel(batch, num_fields, fc_weight.shape[0], nc, ns)
    out = sc(x.T.astype(jnp.int32), fc_weight.T, bias.astype(jnp.float32))
    return out.reshape(batch, 1)


# speedup vs baseline: 3.5003x; 1.0201x over previous
import functools

import jax
import jax.numpy as jnp
from jax import lax
from jax.experimental import pallas as pl
from jax.experimental.pallas import tpu as pltpu
from jax.experimental.pallas import tpu_sc as plsc

_LANES = 16


def _make_sc_kernel(batch, num_fields, vocab, nc, ns):
    nw = nc * ns
    n_per = batch // nw

    n_flat = n_per * num_fields

    def body(xt_hbm, w_hbm, b_hbm, out_hbm, idx_v, vals_v, out_v, bias_v,
             w_sp, sem, sem2, sem3, semw):
        cid = lax.axis_index("c")
        sid = lax.axis_index("s")
        wid = sid * nc + cid
        b0 = wid * n_per

        descs = [
            pltpu.async_copy(
                xt_hbm.at[j, pl.ds(b0, n_per)],
                idx_v.at[pl.ds(j * n_per, n_per)],
                sem2 if j < num_fields // 2 else sem3,
            )
            for j in range(num_fields)
        ]
        pltpu.sync_copy(b_hbm, bias_v.at[pl.ds(0, 1)])

        @pl.when(sid == 0)
        def _():
            pltpu.async_copy(w_hbm, w_sp, semw)

        half = num_fields // 2
        cut = half * n_per
        for d in descs[:half]:
            d.wait()
        ga = pltpu.async_copy(
            w_hbm.at[0].at[idx_v.at[pl.ds(0, cut)]],
            vals_v.at[pl.ds(0, cut)], sem)
        for d in descs[half:]:
            d.wait()

        @pl.when(sid == 0)
        def _():
            pltpu.make_async_copy(w_hbm, w_sp, semw).wait()

        plsc.subcore_barrier()
        gb = pltpu.async_copy(
            w_sp.at[0].at[idx_v.at[pl.ds(cut, n_flat - cut)]],
            vals_v.at[pl.ds(cut, n_flat - cut)], sem3)

        bias_vec = jnp.broadcast_to(bias_v[pl.ds(0, _LANES)][0], (_LANES,))
        nchunk = n_per // _LANES

        ga.wait()

        def chunk_a(c, carry):
            acc = bias_vec
            for j in range(half):
                acc = acc + vals_v[pl.ds(j * n_per + c * _LANES, _LANES)]
            out_v[pl.ds(c * _LANES, _LANES)] = acc
            return carry

        lax.fori_loop(0, nchunk, chunk_a, 0)
        gb.wait()

        def chunk_b(c, carry):
            acc = out_v[pl.ds(c * _LANES, _LANES)]
            for j in range(half, num_fields):
                acc = acc + vals_v[pl.ds(j * n_per + c * _LANES, _LANES)]
            out_v[pl.ds(c * _LANES, _LANES)] = acc
            return carry

        lax.fori_loop(0, nchunk, chunk_b, 0)
        pltpu.sync_copy(out_v, out_hbm.at[pl.ds(b0, n_per)])

    mesh = plsc.VectorSubcoreMesh(
        core_axis_name="c", subcore_axis_name="s", num_cores=nc
    )
    return pl.kernel(
        body,
        out_type=jax.ShapeDtypeStruct((batch,), jnp.float32),
        mesh=mesh,
        compiler_params=pltpu.CompilerParams(
            needs_layout_passes=False, skip_device_barrier=True
        ),
        scratch_types=[
            pltpu.VMEM((n_flat,), jnp.int32),
            pltpu.VMEM((n_flat,), jnp.float32),
            pltpu.VMEM((n_per,), jnp.float32),
            pltpu.VMEM((128,), jnp.float32),
            pltpu.VMEM_SHARED((1, vocab), jnp.float32),
            pltpu.SemaphoreType.DMA,
            pltpu.SemaphoreType.DMA,
            pltpu.SemaphoreType.DMA,
            pltpu.SemaphoreType.DMA,
        ],
    )


@jax.jit
def kernel(x, fc_weight, bias):
    batch, num_fields = x.shape
    info = plsc.get_sparse_core_info()
    nc, ns = info.num_cores, info.num_subcores

    sc = _make_sc_kernel(batch, num_fields, fc_weight.shape[0], nc, ns)
    out = sc(x.T.astype(jnp.int32), fc_weight.T, bias.astype(jnp.float32))
    return out.reshape(batch, 1)
